# baseline (speedup 1.0000x reference)
import jax
import jax.numpy as jnp
from jax.experimental import pallas as pl
from jax.experimental.pallas import tpu as pltpu

_BM = 2304
_NBUF = 3


def _make_pipeline_kernel(nsteps):
    def body(enc_hbm, cb_ref, out_hbm, enc_buf, out_buf, in_sems, out_sems):
        i = pl.program_id(0)

        def in_copy(block, slot):
            return pltpu.make_async_copy(
                enc_hbm.at[pl.ds(block * _BM, _BM), :],
                enc_buf.at[slot],
                in_sems.at[slot],
            )

        def out_copy(block, slot):
            return pltpu.make_async_copy(
                out_buf.at[slot],
                out_hbm.at[pl.ds(block * _BM, _BM), :],
                out_sems.at[slot],
            )

        @pl.when(i == 0)
        def _prologue():
            for b in range(min(_NBUF - 1, nsteps)):
                in_copy(b, b).start()

        fetch = i + _NBUF - 1

        @pl.when(fetch < nsteps)
        def _prefetch():
            in_copy(fetch, jax.lax.rem(fetch, _NBUF)).start()

        slot = jax.lax.rem(i, _NBUF)
        in_copy(i, slot).wait()

        @pl.when(i >= _NBUF)
        def _reclaim():
            out_copy(i - _NBUF, slot).wait()

        out_buf[slot] = jnp.dot(
            enc_buf[slot], cb_ref[...], preferred_element_type=jnp.float32
        )
        out_copy(i, slot).start()

        @pl.when(i == nsteps - 1)
        def _epilogue():
            for d in range(min(_NBUF, nsteps)):
                block = nsteps - 1 - d
                out_copy(block, block % _NBUF).wait()

    return body


def kernel(encodings, codebook):
    m, k = encodings.shape
    _, n = codebook.shape
    nsteps = m // _BM
    return pl.pallas_call(
        _make_pipeline_kernel(nsteps),
        grid=(nsteps,),
        in_specs=[
            pl.BlockSpec(memory_space=pl.ANY),
            pl.BlockSpec((k, n), lambda i: (0, 0)),
        ],
        out_specs=pl.BlockSpec(memory_space=pl.ANY),
        out_shape=jax.ShapeDtypeStruct((m, n), jnp.float32),
        scratch_shapes=[
            pltpu.VMEM((_NBUF, _BM, k), jnp.float32),
            pltpu.VMEM((_NBUF, _BM, n), jnp.float32),
            pltpu.SemaphoreType.DMA((_NBUF,)),
            pltpu.SemaphoreType.DMA((_NBUF,)),
        ],
        compiler_params=pltpu.CompilerParams(
            dimension_semantics=("arbitrary",),
        ),
    )(encodings, codebook)


# speedup vs baseline: 1.0188x; 1.0188x over previous
import jax
import jax.numpy as jnp
from jax.experimental import pallas as pl
from jax.experimental.pallas import tpu as pltpu

_BM = 2304


def _matmul_kernel(enc_ref, cb_ref, out_ref):
    out_ref[...] = jnp.dot(
        enc_ref[...], cb_ref[...], preferred_element_type=jnp.float32
    )


def kernel(encodings, codebook):
    m, k = encodings.shape
    _, n = codebook.shape
    return pl.pallas_call(
        _matmul_kernel,
        grid=(m // _BM,),
        in_specs=[
            pl.BlockSpec((_BM, k), lambda i: (i, 0)),
            pl.BlockSpec((k, n), lambda i: (0, 0)),
        ],
        out_specs=pl.BlockSpec((_BM, n), lambda i: (i, 0)),
        out_shape=jax.ShapeDtypeStruct((m, n), jnp.float32),
        compiler_params=pltpu.CompilerParams(
            dimension_semantics=("arbitrary",),
        ),
    )(encodings, codebook)
